# baseline (speedup 1.0000x reference)
import jax
import jax.numpy as jnp
from jax import lax
from jax.experimental import pallas as pl
from jax.experimental.pallas import tpu as pltpu

N = 64
D = 64
H = 64
E = 512
EPS = 0.0
F32 = jnp.float32
NCORES = 2
BI = N // NCORES


def _mlp(x, W1, b1, W2, b2):
    h = lax.dot_general(x, W1, (((1,), (0,)), ((), ())),
                        preferred_element_type=F32) + b1
    h = jnp.maximum(h, 0.0)
    return lax.dot_general(h, W2, (((1,), (0,)), ((), ())),
                           preferred_element_type=F32) + b2


def _idgnn_kernel(adjT_ref, x_ref,
                  W00a_ref, b00a_ref, W00b_ref, b00b_ref,
                  W01a_ref, b01a_ref, W01b_ref, b01b_ref,
                  W10a_ref, b10a_ref, W10b_ref, b10b_ref,
                  W11a_ref, b11a_ref, W11b_ref, b11b_ref,
                  out_ref, hp_ref, k2t_ref):
    c = pl.program_id(0)
    src = adjT_ref[:, 0:1]
    dst = adjT_ref[:, 1:2]
    iota_e = lax.broadcasted_iota(jnp.int32, (E, N), 1)
    o_src = (src == iota_e).astype(F32)
    o_dst = (dst == iota_e).astype(F32)
    acnt = lax.dot_general(o_src, o_dst, (((0,), (0,)), ((), ())),
                           preferred_element_type=F32)
    A = jnp.minimum(acnt, 1.0)

    ident = (lax.broadcasted_iota(jnp.int32, (BI, BI), 0)
             == lax.broadcasted_iota(jnp.int32, (BI, BI), 1)).astype(F32)
    glob_i = lax.broadcasted_iota(jnp.int32, (BI, N), 0) + c * BI
    dsel = (glob_i == lax.broadcasted_iota(jnp.int32, (BI, N), 1)).astype(F32)
    iota_n = lax.broadcasted_iota(jnp.int32, (BI, N), 1).astype(F32)

    def rows_sub(M):
        return lax.dot_general(dsel, M, (((1,), (0,)), ((), ())),
                               preferred_element_type=F32)

    Asub = rows_sub(A)
    k2t_ref[...] = lax.dot_general(A, Asub, (((0,), (1,)), ((), ())),
                                   preferred_element_type=F32)

    x = x_ref[...]

    def scan(Dg, H1):
        def step(t, carry):
            Dg, H1 = carry
            jrow = k2t_ref[pl.ds(t, 1), :]
            jcol = jnp.sum(ident * jrow, axis=1, keepdims=True)
            jc = jnp.minimum(jcol, float(N - 1))
            hit64 = jcol >= float(N) - 0.5
            Mg = (iota_n == jc).astype(F32)
            Ms = Mg * (1.0 - hit64.astype(F32))
            R = lax.dot_general(Mg, A, (((1,), (0,)), ((), ())),
                                preferred_element_type=F32)
            r = jnp.sum(R * dsel, axis=1, keepdims=True)
            hp = hp_ref[...]
            P = jnp.sum(R[:, :, None] * hp, axis=1)
            S = P + r * (H1 - Dg)
            hp_ref[...] = jnp.where(Ms[:, :, None] > 0.5, S[:, None, :], hp)
            dghit = jnp.sum(Ms * dsel, axis=1, keepdims=True) > 0.5
            Dg = jnp.where(dghit, S, Dg)
            H1 = jnp.where(hit64, S, H1)
            return (Dg, H1)
        return lax.fori_loop(0, N, step, (Dg, H1))

    H0 = _mlp(x, W00a_ref[...], b00a_ref[...], W00b_ref[...], b00b_ref[...])
    H1f = _mlp(x, W01a_ref[...], b01a_ref[...], W01b_ref[...], b01b_ref[...])
    H0sub = rows_sub(H0)
    H1sub = rows_sub(H1f)
    hp_ref[...] = jnp.broadcast_to(H0[None], (BI, N, H))
    Dg, _ = scan(H0sub, H1sub)
    hj1 = hp_ref[...] + H0[None]
    y_diag = Dg + H0sub

    h0b = _mlp(hj1.reshape(BI * N, D), W10a_ref[...], b10a_ref[...],
               W10b_ref[...], b10b_ref[...]).reshape(BI, N, H)
    H1b = _mlp(y_diag, W11a_ref[...], b11a_ref[...],
               W11b_ref[...], b11b_ref[...])
    hp_ref[...] = h0b
    Dg2_init = jnp.sum(dsel[:, :, None] * h0b, axis=1)
    Dg2, _ = scan(Dg2_init, H1b)
    out_ref[...] = Dg2 + (1.0 + EPS) * Dg2_init


def kernel(x, adj1, W00a, b00a, W00b, b00b, W01a, b01a, W01b, b01b,
           W10a, b10a, W10b, b10b, W11a, b11a, W11b, b11b):
    adjT = adj1.astype(jnp.int32).T
    biases = [b.reshape(1, H) for b in
              (b00a, b00b, b01a, b01b, b10a, b10b, b11a, b11b)]
    (b00a2, b00b2, b01a2, b01b2, b10a2, b10b2, b11a2, b11b2) = biases
    full = lambda shape: pl.BlockSpec(shape, lambda c: (0,) * len(shape))
    in_specs = [full((E, 2)), full((N, D))]
    for _ in range(4):
        in_specs += [full((D, H)), full((1, H)), full((H, H)), full((1, H))]
    return pl.pallas_call(
        _idgnn_kernel,
        grid=(NCORES,),
        in_specs=in_specs,
        out_specs=pl.BlockSpec((BI, H), lambda c: (c, 0)),
        out_shape=jax.ShapeDtypeStruct((N, H), F32),
        scratch_shapes=[pltpu.VMEM((BI, N, H), F32),
                        pltpu.VMEM((N, BI), F32)],
        compiler_params=pltpu.CompilerParams(
            dimension_semantics=("parallel",)),
    )(adjT, x, W00a, b00a2, W00b, b00b2, W01a, b01a2, W01b, b01b2,
      W10a, b10a2, W10b, b10b2, W11a, b11a2, W11b, b11b2)


# speedup vs baseline: 2.2312x; 2.2312x over previous
import jax
import jax.numpy as jnp
from jax import lax
from jax.experimental import pallas as pl
from jax.experimental.pallas import tpu as pltpu

N = 64
D = 64
H = 64
E = 512
EPS = 0.0
F32 = jnp.float32


def _mlp_t(xt, W1t, b1, W2t, b2):
    h = lax.dot_general(W1t, xt, (((1,), (0,)), ((), ())),
                        preferred_element_type=F32) + b1
    h = jnp.maximum(h, 0.0)
    return lax.dot_general(W2t, h, (((1,), (0,)), ((), ())),
                           preferred_element_type=F32) + b2


def _idgnn_kernel(adjT_ref, xt_ref,
                  W00a_ref, b00a_ref, W00b_ref, b00b_ref,
                  W01a_ref, b01a_ref, W01b_ref, b01b_ref,
                  W10a_ref, b10a_ref, W10b_ref, b10b_ref,
                  W11a_ref, b11a_ref, W11b_ref, b11b_ref,
                  out_ref, hp_ref, hp0_ref, k2t_ref):
    src = adjT_ref[:, 0:1]
    dst = adjT_ref[:, 1:2]
    iota_e = lax.broadcasted_iota(jnp.int32, (E, N), 1)
    o_src = (src == iota_e).astype(F32)
    o_dst = (dst == iota_e).astype(F32)
    acnt = lax.dot_general(o_src, o_dst, (((0,), (0,)), ((), ())),
                           preferred_element_type=F32)
    A = jnp.minimum(acnt, 1.0)

    k2t_ref[...] = lax.dot_general(A, A, (((0,), (1,)), ((), ())),
                                   preferred_element_type=F32)

    iota_ns = lax.broadcasted_iota(jnp.int32, (N, N), 0).astype(F32)
    iota_ir = lax.broadcasted_iota(jnp.int32, (1, N), 1).astype(F32)
    ident = (lax.broadcasted_iota(jnp.int32, (N, N), 0)
             == lax.broadcasted_iota(jnp.int32, (N, N), 1)).astype(F32)

    def scan(DgT, H1T):
        def step(t, carry):
            DgT, H1T = carry
            jrow = k2t_ref[pl.ds(t, 1), :]
            jcs = jnp.minimum(jrow, float(N - 1))
            hit64 = jrow >= float(N) - 0.5
            MgT = (iota_ns == jcs).astype(F32)
            MsT = MgT * (1.0 - hit64.astype(F32))
            RT = lax.dot_general(A, MgT, (((0,), (0,)), ((), ())),
                                 preferred_element_type=F32)
            rT = jnp.sum(MgT * A, axis=0, keepdims=True)
            hp = hp_ref[...]
            P = jnp.sum(RT[None, :, :] * hp, axis=1)
            S = P + rT * (H1T - DgT)
            hp_ref[...] = jnp.where(MsT[None, :, :] > 0.5, S[:, None, :], hp)
            dghit = jrow == iota_ir
            DgT = jnp.where(dghit, S, DgT)
            H1T = jnp.where(hit64, S, H1T)
            return (DgT, H1T)
        return lax.fori_loop(0, N, step, (DgT, H1T))

    xt = xt_ref[...]
    H0T = _mlp_t(xt, W00a_ref[...], b00a_ref[...], W00b_ref[...],
                 b00b_ref[...])
    H1T0 = _mlp_t(xt, W01a_ref[...], b01a_ref[...], W01b_ref[...],
                  b01b_ref[...])
    hp_init = jnp.broadcast_to(H0T[:, :, None], (H, N, N))
    hp_ref[...] = hp_init
    hp0_ref[...] = hp_init
    DgT, _ = scan(H0T, H1T0)
    hj1t = hp_ref[...] + hp0_ref[...]
    y_diag_t = DgT + H0T

    h0bt = _mlp_t(hj1t.reshape(H, N * N), W10a_ref[...], b10a_ref[...],
                  W10b_ref[...], b10b_ref[...]).reshape(H, N, N)
    H1bT = _mlp_t(y_diag_t, W11a_ref[...], b11a_ref[...],
                  W11b_ref[...], b11b_ref[...])
    hp_ref[...] = h0bt
    Dg2iT = jnp.sum(h0bt * ident[None, :, :], axis=1)
    Dg2T, _ = scan(Dg2iT, H1bT)
    out_ref[...] = Dg2T + (1.0 + EPS) * Dg2iT


def kernel(x, adj1, W00a, b00a, W00b, b00b, W01a, b01a, W01b, b01b,
           W10a, b10a, W10b, b10b, W11a, b11a, W11b, b11b):
    adjT = adj1.astype(jnp.int32).T
    wts = [w.T for w in (W00a, W00b, W01a, W01b, W10a, W10b, W11a, W11b)]
    bs = [b.reshape(H, 1) for b in
          (b00a, b00b, b01a, b01b, b10a, b10b, b11a, b11b)]
    out_t = pl.pallas_call(
        _idgnn_kernel,
        out_shape=jax.ShapeDtypeStruct((H, N), F32),
        scratch_shapes=[pltpu.VMEM((H, N, N), F32),
                        pltpu.VMEM((H, N, N), F32),
                        pltpu.VMEM((N, N), F32)],
    )(adjT, x.T, wts[0], bs[0], wts[1], bs[1], wts[2], bs[2], wts[3], bs[3],
      wts[4], bs[4], wts[5], bs[5], wts[6], bs[6], wts[7], bs[7])
    return out_t.T


# speedup vs baseline: 2.5214x; 1.1301x over previous
import jax
import jax.numpy as jnp
from jax import lax
from jax.experimental import pallas as pl
from jax.experimental.pallas import tpu as pltpu

N = 64
D = 64
H = 64
E = 512
EPS = 0.0
F32 = jnp.float32
H2 = H // 2


def _mlp_t(xt, W1t, b1, W2t, b2):
    h = lax.dot_general(W1t, xt, (((1,), (0,)), ((), ())),
                        preferred_element_type=F32) + b1
    h = jnp.maximum(h, 0.0)
    return lax.dot_general(W2t, h, (((1,), (0,)), ((), ())),
                           preferred_element_type=F32) + b2


def _dup(m):
    return jnp.concatenate([m, m], axis=-1)


def _pack(m):
    return jnp.concatenate([m[:H2], m[H2:]], axis=-1)


def _unpack(m):
    return jnp.concatenate([m[:, :N], m[:, N:]], axis=0)


def _idgnn_kernel(adjT_ref, xt_ref,
                  W00a_ref, b00a_ref, W00b_ref, b00b_ref,
                  W01a_ref, b01a_ref, W01b_ref, b01b_ref,
                  W10a_ref, b10a_ref, W10b_ref, b10b_ref,
                  W11a_ref, b11a_ref, W11b_ref, b11b_ref,
                  out_ref, hp_ref, hp0_ref, k2t_ref):
    src = adjT_ref[:, 0:1]
    dst = adjT_ref[:, 1:2]
    iota_e = lax.broadcasted_iota(jnp.int32, (E, N), 1)
    o_src = (src == iota_e).astype(F32)
    o_dst = (dst == iota_e).astype(F32)
    acnt = lax.dot_general(o_src, o_dst, (((0,), (0,)), ((), ())),
                           preferred_element_type=F32)
    A = jnp.minimum(acnt, 1.0)

    k2t_ref[...] = lax.dot_general(A, A, (((0,), (1,)), ((), ())),
                                   preferred_element_type=F32)

    iota_ns = lax.broadcasted_iota(jnp.int32, (N, N), 0).astype(F32)
    iota_ir = lax.broadcasted_iota(jnp.int32, (1, N), 1).astype(F32)
    ident = (lax.broadcasted_iota(jnp.int32, (N, N), 0)
             == lax.broadcasted_iota(jnp.int32, (N, N), 1)).astype(F32)

    def scan(DgP, H1P):
        def step(t, carry):
            DgP, H1P = carry
            jrow = k2t_ref[pl.ds(t, 1), :]
            jcs = jnp.minimum(jrow, float(N - 1))
            hit64 = jrow >= float(N) - 0.5
            MgT = (iota_ns == jcs).astype(F32)
            MsT = MgT * (1.0 - hit64.astype(F32))
            RT = lax.dot_general(A, MgT, (((0,), (0,)), ((), ())),
                                 preferred_element_type=F32)
            rT = jnp.sum(MgT * A, axis=0, keepdims=True)
            RTd = _dup(RT)
            Msd = _dup(MsT)
            hp = hp_ref[...]
            P = jnp.sum(RTd[None, :, :] * hp, axis=1)
            S = P + _dup(rT) * (H1P - DgP)
            hp_ref[...] = jnp.where(Msd[None, :, :] > 0.5, S[:, None, :], hp)
            dghit = _dup((jrow == iota_ir).astype(F32))
            DgP = jnp.where(dghit > 0.5, S, DgP)
            H1P = jnp.where(_dup(hit64.astype(F32)) > 0.5, S, H1P)
            return (DgP, H1P)
        return lax.fori_loop(0, N, step, (DgP, H1P))

    xt = xt_ref[...]
    H0T = _mlp_t(xt, W00a_ref[...], b00a_ref[...], W00b_ref[...],
                 b00b_ref[...])
    H1T0 = _mlp_t(xt, W01a_ref[...], b01a_ref[...], W01b_ref[...],
                  b01b_ref[...])
    hp_init = jnp.concatenate(
        [jnp.broadcast_to(H0T[:H2, :, None], (H2, N, N)),
         jnp.broadcast_to(H0T[H2:, :, None], (H2, N, N))], axis=2)
    hp_ref[...] = hp_init
    hp0_ref[...] = hp_init
    DgP, _ = scan(_pack(H0T), _pack(H1T0))
    hj1p = hp_ref[...] + hp0_ref[...]
    hj1t = jnp.concatenate([hj1p[:, :, :N], hj1p[:, :, N:]], axis=0)
    y_diag_p = DgP + _pack(H0T)

    h0bt = _mlp_t(hj1t.reshape(H, N * N), W10a_ref[...], b10a_ref[...],
                  W10b_ref[...], b10b_ref[...]).reshape(H, N, N)
    H1bT = _mlp_t(_unpack(y_diag_p), W11a_ref[...], b11a_ref[...],
                  W11b_ref[...], b11b_ref[...])
    h0bp = jnp.concatenate([h0bt[:H2], h0bt[H2:]], axis=2)
    hp_ref[...] = h0bp
    Dg2iP = jnp.sum(h0bp * _dup(ident)[None, :, :], axis=1)
    Dg2P, _ = scan(Dg2iP, _pack(H1bT))
    outp = Dg2P + (1.0 + EPS) * Dg2iP
    out_ref[...] = _unpack(outp)


def kernel(x, adj1, W00a, b00a, W00b, b00b, W01a, b01a, W01b, b01b,
           W10a, b10a, W10b, b10b, W11a, b11a, W11b, b11b):
    adjT = adj1.astype(jnp.int32).T
    wts = [w.T for w in (W00a, W00b, W01a, W01b, W10a, W10b, W11a, W11b)]
    bs = [b.reshape(H, 1) for b in
          (b00a, b00b, b01a, b01b, b10a, b10b, b11a, b11b)]
    out_t = pl.pallas_call(
        _idgnn_kernel,
        out_shape=jax.ShapeDtypeStruct((H, N), F32),
        scratch_shapes=[pltpu.VMEM((H2, N, 2 * N), F32),
                        pltpu.VMEM((H2, N, 2 * N), F32),
                        pltpu.VMEM((N, N), F32)],
    )(adjT, x.T, wts[0], bs[0], wts[1], bs[1], wts[2], bs[2], wts[3], bs[3],
      wts[4], bs[4], wts[5], bs[5], wts[6], bs[6], wts[7], bs[7])
    return out_t.T


# speedup vs baseline: 4.5772x; 1.8153x over previous
import jax
import jax.numpy as jnp
from jax import lax
from jax.experimental import pallas as pl
from jax.experimental.pallas import tpu as pltpu

N = 64
D = 64
H = 64
E = 512
EPS = 0.0
F32 = jnp.float32
H2 = H // 2


def _mlp_t(xt, W1t, b1, W2t, b2):
    h = lax.dot_general(W1t, xt, (((1,), (0,)), ((), ())),
                        preferred_element_type=F32) + b1
    h = jnp.maximum(h, 0.0)
    return lax.dot_general(W2t, h, (((1,), (0,)), ((), ())),
                           preferred_element_type=F32) + b2


def _dup(m):
    return jnp.concatenate([m, m], axis=-1)


def _pack(m):
    return jnp.concatenate([m[:H2], m[H2:]], axis=-1)


def _unpack(m):
    return jnp.concatenate([m[:, :N], m[:, N:]], axis=0)


def _idgnn_kernel(adjT_ref, xt_ref,
                  W00a_ref, b00a_ref, W00b_ref, b00b_ref,
                  W01a_ref, b01a_ref, W01b_ref, b01b_ref,
                  W10a_ref, b10a_ref, W10b_ref, b10b_ref,
                  W11a_ref, b11a_ref, W11b_ref, b11b_ref,
                  out_ref, hp_ref, hp0_ref, k2t_ref):
    src = adjT_ref[:, 0:1]
    dst = adjT_ref[:, 1:2]
    iota_e = lax.broadcasted_iota(jnp.int32, (E, N), 1)
    o_src = (src == iota_e).astype(F32)
    o_dst = (dst == iota_e).astype(F32)
    acnt = lax.dot_general(o_src, o_dst, (((0,), (0,)), ((), ())),
                           preferred_element_type=F32)
    A = jnp.minimum(acnt, 1.0)

    k2t = lax.dot_general(A, A, (((0,), (1,)), ((), ())),
                          preferred_element_type=F32)
    k2t_ref[0:N, :] = _dup(k2t)
    k2t_ref[N:, :] = jnp.zeros((8, 2 * N), F32)

    iota_ns = lax.broadcasted_iota(jnp.int32, (N, 2 * N), 0).astype(F32)
    iota_ir = _dup(lax.broadcasted_iota(jnp.int32, (1, N), 1).astype(F32))
    ident = (lax.broadcasted_iota(jnp.int32, (N, N), 0)
             == lax.broadcasted_iota(jnp.int32, (N, N), 1)).astype(F32)
    identd = _dup(ident)
    Ad = _dup(A)

    def premask(t):
        jrow = k2t_ref[pl.ds(t, 1), :]
        jcs = jnp.minimum(jrow, float(N - 1))
        hit64 = (jrow >= float(N) - 0.5).astype(F32)
        MgTd = (iota_ns == jcs).astype(F32)
        MsTd = MgTd * (1.0 - hit64)
        RTd = lax.dot_general(A, MgTd, (((0,), (0,)), ((), ())),
                              preferred_element_type=F32)
        rTd = jnp.sum(MgTd * Ad, axis=0, keepdims=True)
        dghit = (jrow == iota_ir).astype(F32)
        return (RTd, MsTd, rTd, dghit, hit64)

    def scan(DgP, H1P):
        def step(t, carry):
            DgP, H1P, pre = carry
            RTd, MsTd, rTd, dghit, hit64 = pre
            pre_next = premask(t + 1)
            hp = hp_ref[...]
            P = jnp.sum(RTd[None, :, :] * hp, axis=1)
            S = P + rTd * (H1P - DgP)
            hp_ref[...] = jnp.where(MsTd[None, :, :] > 0.5,
                                    S[:, None, :], hp)
            DgP = jnp.where(dghit > 0.5, S, DgP)
            H1P = jnp.where(hit64 > 0.5, S, H1P)
            return (DgP, H1P, pre_next)
        DgP, H1P, _ = lax.fori_loop(0, N, step, (DgP, H1P, premask(0)))
        return DgP, H1P

    xt = xt_ref[...]
    H0T = _mlp_t(xt, W00a_ref[...], b00a_ref[...], W00b_ref[...],
                 b00b_ref[...])
    H1T0 = _mlp_t(xt, W01a_ref[...], b01a_ref[...], W01b_ref[...],
                  b01b_ref[...])
    hp_init = jnp.concatenate(
        [jnp.broadcast_to(H0T[:H2, :, None], (H2, N, N)),
         jnp.broadcast_to(H0T[H2:, :, None], (H2, N, N))], axis=2)
    hp_ref[...] = hp_init
    hp0_ref[...] = hp_init
    DgP, _ = scan(_pack(H0T), _pack(H1T0))
    hj1p = hp_ref[...] + hp0_ref[...]
    hj1t = jnp.concatenate([hj1p[:, :, :N], hj1p[:, :, N:]], axis=0)
    y_diag_p = DgP + _pack(H0T)

    h0bt = _mlp_t(hj1t.reshape(H, N * N), W10a_ref[...], b10a_ref[...],
                  W10b_ref[...], b10b_ref[...]).reshape(H, N, N)
    H1bT = _mlp_t(_unpack(y_diag_p), W11a_ref[...], b11a_ref[...],
                  W11b_ref[...], b11b_ref[...])
    h0bp = jnp.concatenate([h0bt[:H2], h0bt[H2:]], axis=2)
    hp_ref[...] = h0bp
    Dg2iP = jnp.sum(h0bp * identd[None, :, :], axis=1)
    Dg2P, _ = scan(Dg2iP, _pack(H1bT))
    outp = Dg2P + (1.0 + EPS) * Dg2iP
    out_ref[...] = _unpack(outp)


def kernel(x, adj1, W00a, b00a, W00b, b00b, W01a, b01a, W01b, b01b,
           W10a, b10a, W10b, b10b, W11a, b11a, W11b, b11b):
    adjT = adj1.astype(jnp.int32).T
    wts = [w.T for w in (W00a, W00b, W01a, W01b, W10a, W10b, W11a, W11b)]
    bs = [b.reshape(H, 1) for b in
          (b00a, b00b, b01a, b01b, b10a, b10b, b11a, b11b)]
    out_t = pl.pallas_call(
        _idgnn_kernel,
        out_shape=jax.ShapeDtypeStruct((H, N), F32),
        scratch_shapes=[pltpu.VMEM((H2, N, 2 * N), F32),
                        pltpu.VMEM((H2, N, 2 * N), F32),
                        pltpu.VMEM((N + 8, 2 * N), F32)],
    )(adjT, x.T, wts[0], bs[0], wts[1], bs[1], wts[2], bs[2], wts[3], bs[3],
      wts[4], bs[4], wts[5], bs[5], wts[6], bs[6], wts[7], bs[7])
    return out_t.T


# speedup vs baseline: 4.6618x; 1.0185x over previous
import jax
import jax.numpy as jnp
from jax import lax
from jax.experimental import pallas as pl
from jax.experimental.pallas import tpu as pltpu

N = 64
D = 64
H = 64
E = 512
EPS = 0.0
F32 = jnp.float32
H2 = H // 2


def _mlp_t(xt, W1t, b1, W2t, b2):
    h = lax.dot_general(W1t, xt, (((1,), (0,)), ((), ())),
                        preferred_element_type=F32) + b1
    h = jnp.maximum(h, 0.0)
    return lax.dot_general(W2t, h, (((1,), (0,)), ((), ())),
                           preferred_element_type=F32) + b2


def _dup(m):
    return jnp.concatenate([m, m], axis=-1)


def _pack(m):
    return jnp.concatenate([m[:H2], m[H2:]], axis=-1)


def _unpack(m):
    return jnp.concatenate([m[:, :N], m[:, N:]], axis=0)


def _idgnn_kernel(adjT_ref, xt_ref,
                  W00a_ref, b00a_ref, W00b_ref, b00b_ref,
                  W01a_ref, b01a_ref, W01b_ref, b01b_ref,
                  W10a_ref, b10a_ref, W10b_ref, b10b_ref,
                  W11a_ref, b11a_ref, W11b_ref, b11b_ref,
                  out_ref, hp_ref, hp0_ref, k2t_ref):
    src = adjT_ref[:, 0:1]
    dst = adjT_ref[:, 1:2]
    iota_e = lax.broadcasted_iota(jnp.int32, (E, N), 1)
    o_src = (src == iota_e).astype(F32)
    o_dst = (dst == iota_e).astype(F32)
    acnt = lax.dot_general(o_src, o_dst, (((0,), (0,)), ((), ())),
                           preferred_element_type=F32)
    A = jnp.minimum(acnt, 1.0)

    k2t = lax.dot_general(A, A, (((0,), (1,)), ((), ())),
                          preferred_element_type=F32)
    k2t_ref[0:N, :] = _dup(k2t)
    k2t_ref[N:, :] = jnp.zeros((8, 2 * N), F32)

    iota_ns = lax.broadcasted_iota(jnp.int32, (N, 2 * N), 0).astype(F32)
    iota_ir = _dup(lax.broadcasted_iota(jnp.int32, (1, N), 1).astype(F32))
    ident = (lax.broadcasted_iota(jnp.int32, (N, N), 0)
             == lax.broadcasted_iota(jnp.int32, (N, N), 1)).astype(F32)
    identd = _dup(ident)
    Ad = _dup(A)

    def premask(t):
        jrow = k2t_ref[pl.ds(t, 1), :]
        jcs = jnp.minimum(jrow, float(N - 1))
        hit64 = (jrow >= float(N) - 0.5).astype(F32)
        MgTd = (iota_ns == jcs).astype(F32)
        MsTd = MgTd * (1.0 - hit64)
        RTd = lax.dot_general(A, MgTd, (((0,), (0,)), ((), ())),
                              preferred_element_type=F32)
        rTd = jnp.sum(MgTd * Ad, axis=0, keepdims=True)
        dghit = (jrow == iota_ir).astype(F32)
        return (RTd, MsTd, rTd, dghit, hit64)

    def scan(DgP, H1P):
        def step(t, carry):
            DgP, H1P, pre = carry
            RTd, MsTd, rTd, dghit, hit64 = pre
            pre_next = premask(t + 1)
            hp = hp_ref[...]
            P = jnp.sum(RTd[None, :, :] * hp, axis=1)
            S = P + rTd * (H1P - DgP)
            hp_ref[...] = jnp.where(MsTd[None, :, :] > 0.5,
                                    S[:, None, :], hp)
            DgP = jnp.where(dghit > 0.5, S, DgP)
            H1P = jnp.where(hit64 > 0.5, S, H1P)
            return (DgP, H1P, pre_next)
        DgP, H1P, _ = lax.fori_loop(0, N, step, (DgP, H1P, premask(0)),
                                    unroll=2)
        return DgP, H1P

    xt = xt_ref[...]
    H0T = _mlp_t(xt, W00a_ref[...], b00a_ref[...], W00b_ref[...],
                 b00b_ref[...])
    H1T0 = _mlp_t(xt, W01a_ref[...], b01a_ref[...], W01b_ref[...],
                  b01b_ref[...])
    hp_init = jnp.concatenate(
        [jnp.broadcast_to(H0T[:H2, :, None], (H2, N, N)),
         jnp.broadcast_to(H0T[H2:, :, None], (H2, N, N))], axis=2)
    hp_ref[...] = hp_init
    hp0_ref[...] = hp_init
    DgP, _ = scan(_pack(H0T), _pack(H1T0))
    hj1p = hp_ref[...] + hp0_ref[...]
    hj1t = jnp.concatenate([hj1p[:, :, :N], hj1p[:, :, N:]], axis=0)
    y_diag_p = DgP + _pack(H0T)

    h0bt = _mlp_t(hj1t.reshape(H, N * N), W10a_ref[...], b10a_ref[...],
                  W10b_ref[...], b10b_ref[...]).reshape(H, N, N)
    H1bT = _mlp_t(_unpack(y_diag_p), W11a_ref[...], b11a_ref[...],
                  W11b_ref[...], b11b_ref[...])
    h0bp = jnp.concatenate([h0bt[:H2], h0bt[H2:]], axis=2)
    hp_ref[...] = h0bp
    Dg2iP = jnp.sum(h0bp * identd[None, :, :], axis=1)
    Dg2P, _ = scan(Dg2iP, _pack(H1bT))
    outp = Dg2P + (1.0 + EPS) * Dg2iP
    out_ref[...] = _unpack(outp)


def kernel(x, adj1, W00a, b00a, W00b, b00b, W01a, b01a, W01b, b01b,
           W10a, b10a, W10b, b10b, W11a, b11a, W11b, b11b):
    adjT = adj1.astype(jnp.int32).T
    wts = [w.T for w in (W00a, W00b, W01a, W01b, W10a, W10b, W11a, W11b)]
    bs = [b.reshape(H, 1) for b in
          (b00a, b00b, b01a, b01b, b10a, b10b, b11a, b11b)]
    out_t = pl.pallas_call(
        _idgnn_kernel,
        out_shape=jax.ShapeDtypeStruct((H, N), F32),
        scratch_shapes=[pltpu.VMEM((H2, N, 2 * N), F32),
                        pltpu.VMEM((H2, N, 2 * N), F32),
                        pltpu.VMEM((N + 8, 2 * N), F32)],
    )(adjT, x.T, wts[0], bs[0], wts[1], bs[1], wts[2], bs[2], wts[3], bs[3],
      wts[4], bs[4], wts[5], bs[5], wts[6], bs[6], wts[7], bs[7])
    return out_t.T


# speedup vs baseline: 4.6842x; 1.0048x over previous
import jax
import jax.numpy as jnp
from jax import lax
from jax.experimental import pallas as pl
from jax.experimental.pallas import tpu as pltpu

N = 64
D = 64
H = 64
E = 512
EPS = 0.0
F32 = jnp.float32
H2 = H // 2


def _mlp_t(xt, W1t, b1, W2t, b2):
    h = lax.dot_general(W1t, xt, (((1,), (0,)), ((), ())),
                        preferred_element_type=F32) + b1
    h = jnp.maximum(h, 0.0)
    return lax.dot_general(W2t, h, (((1,), (0,)), ((), ())),
                           preferred_element_type=F32) + b2


def _dup(m):
    return jnp.concatenate([m, m], axis=-1)


def _pack(m):
    return jnp.concatenate([m[:H2], m[H2:]], axis=-1)


def _unpack(m):
    return jnp.concatenate([m[:, :N], m[:, N:]], axis=0)


def _idgnn_kernel(adjT_ref, xt_ref,
                  W00a_ref, b00a_ref, W00b_ref, b00b_ref,
                  W01a_ref, b01a_ref, W01b_ref, b01b_ref,
                  W10a_ref, b10a_ref, W10b_ref, b10b_ref,
                  W11a_ref, b11a_ref, W11b_ref, b11b_ref,
                  out_ref, hp_ref, hp0_ref, k2t_ref):
    src = adjT_ref[:, 0:1]
    dst = adjT_ref[:, 1:2]
    iota_e = lax.broadcasted_iota(jnp.int32, (E, N), 1)
    o_src = (src == iota_e).astype(F32)
    o_dst = (dst == iota_e).astype(F32)
    acnt = lax.dot_general(o_src, o_dst, (((0,), (0,)), ((), ())),
                           preferred_element_type=F32)
    A = jnp.minimum(acnt, 1.0)

    k2t = lax.dot_general(A, A, (((0,), (1,)), ((), ())),
                          preferred_element_type=F32)
    k2t_ref[0:N, :] = _dup(k2t)
    k2t_ref[N:, :] = jnp.zeros((8, 2 * N), F32)

    iota_ns = lax.broadcasted_iota(jnp.int32, (N, 2 * N), 0).astype(F32)
    iota_ir = _dup(lax.broadcasted_iota(jnp.int32, (1, N), 1).astype(F32))
    ident = (lax.broadcasted_iota(jnp.int32, (N, N), 0)
             == lax.broadcasted_iota(jnp.int32, (N, N), 1)).astype(F32)
    identd = _dup(ident)
    Ad = _dup(A)

    def premask(t):
        jrow = k2t_ref[pl.ds(t, 1), :]
        jcs = jnp.minimum(jrow, float(N - 1))
        hit64 = (jrow >= float(N) - 0.5).astype(F32)
        MgTd = (iota_ns == jcs).astype(F32)
        MsTd = MgTd * (1.0 - hit64)
        RTd = lax.dot_general(A, MgTd, (((0,), (0,)), ((), ())),
                              preferred_element_type=F32)
        rTd = jnp.sum(MgTd * Ad, axis=0, keepdims=True)
        dghit = (jrow == iota_ir).astype(F32)
        return (RTd, MsTd, rTd, dghit, hit64)

    def scan(DgP, H1P):
        def step(t, carry):
            DgP, H1P, pre = carry
            RTd, MsTd, rTd, dghit, hit64 = pre
            pre_next = premask(t + 1)
            hp = hp_ref[...]
            P = jnp.sum(RTd[None, :, :] * hp, axis=1)
            S = P + rTd * (H1P - DgP)
            hp_ref[...] = jnp.where(MsTd[None, :, :] > 0.5,
                                    S[:, None, :], hp)
            DgP = jnp.where(dghit > 0.5, S, DgP)
            H1P = jnp.where(hit64 > 0.5, S, H1P)
            return (DgP, H1P, pre_next)
        DgP, H1P, _ = lax.fori_loop(0, N, step, (DgP, H1P, premask(0)),
                                    unroll=4)
        return DgP, H1P

    xt = xt_ref[...]
    H0T = _mlp_t(xt, W00a_ref[...], b00a_ref[...], W00b_ref[...],
                 b00b_ref[...])
    H1T0 = _mlp_t(xt, W01a_ref[...], b01a_ref[...], W01b_ref[...],
                  b01b_ref[...])
    hp_init = jnp.concatenate(
        [jnp.broadcast_to(H0T[:H2, :, None], (H2, N, N)),
         jnp.broadcast_to(H0T[H2:, :, None], (H2, N, N))], axis=2)
    hp_ref[...] = hp_init
    hp0_ref[...] = hp_init
    DgP, _ = scan(_pack(H0T), _pack(H1T0))
    hj1p = hp_ref[...] + hp0_ref[...]
    hj1t = jnp.concatenate([hj1p[:, :, :N], hj1p[:, :, N:]], axis=0)
    y_diag_p = DgP + _pack(H0T)

    h0bt = _mlp_t(hj1t.reshape(H, N * N), W10a_ref[...], b10a_ref[...],
                  W10b_ref[...], b10b_ref[...]).reshape(H, N, N)
    H1bT = _mlp_t(_unpack(y_diag_p), W11a_ref[...], b11a_ref[...],
                  W11b_ref[...], b11b_ref[...])
    h0bp = jnp.concatenate([h0bt[:H2], h0bt[H2:]], axis=2)
    hp_ref[...] = h0bp
    Dg2iP = jnp.sum(h0bp * identd[None, :, :], axis=1)
    Dg2P, _ = scan(Dg2iP, _pack(H1bT))
    outp = Dg2P + (1.0 + EPS) * Dg2iP
    out_ref[...] = _unpack(outp)


def kernel(x, adj1, W00a, b00a, W00b, b00b, W01a, b01a, W01b, b01b,
           W10a, b10a, W10b, b10b, W11a, b11a, W11b, b11b):
    adjT = adj1.astype(jnp.int32).T
    wts = [w.T for w in (W00a, W00b, W01a, W01b, W10a, W10b, W11a, W11b)]
    bs = [b.reshape(H, 1) for b in
          (b00a, b00b, b01a, b01b, b10a, b10b, b11a, b11b)]
    out_t = pl.pallas_call(
        _idgnn_kernel,
        out_shape=jax.ShapeDtypeStruct((H, N), F32),
        scratch_shapes=[pltpu.VMEM((H2, N, 2 * N), F32),
                        pltpu.VMEM((H2, N, 2 * N), F32),
                        pltpu.VMEM((N + 8, 2 * N), F32)],
    )(adjT, x.T, wts[0], bs[0], wts[1], bs[1], wts[2], bs[2], wts[3], bs[3],
      wts[4], bs[4], wts[5], bs[5], wts[6], bs[6], wts[7], bs[7])
    return out_t.T
